# speedup vs baseline: 12.2219x; 12.2219x over previous
import functools

import jax
import jax.numpy as jnp
from jax.experimental import pallas as pl
from jax.experimental.pallas import tpu as pltpu

K_TOP = 64
_BISECT_ITERS = 30


def _encode_kernel(x_ref, w_ref, benc_ref, bdec_ref, z_ref):
    xc = x_ref[...] - bdec_ref[...]
    acc = jnp.dot(xc, w_ref[...], preferred_element_type=jnp.float32)
    z_ref[...] = jnp.maximum(acc + benc_ref[...], 0.0)


def _mask_kernel(z_ref, out_ref, *, k):
    z = z_ref[...]
    row_max = jnp.max(z, axis=1, keepdims=True)
    lo = jnp.zeros_like(row_max)
    hi = row_max * 1.000001 + 1e-30

    def body(_, carry):
        lo, hi = carry
        mid = 0.5 * (lo + hi)
        cnt = jnp.sum((z >= mid).astype(jnp.float32), axis=1, keepdims=True)
        pred = cnt >= k
        return jnp.where(pred, mid, lo), jnp.where(pred, hi, mid)

    lo, hi = jax.lax.fori_loop(0, _BISECT_ITERS, body, (lo, hi))
    out_ref[...] = jnp.where(z >= lo, z, 0.0)


def _decode_kernel(z_ref, w_ref, bdec_ref, out_ref):
    j = pl.program_id(1)

    @pl.when(j == 0)
    def _init():
        out_ref[...] = jnp.broadcast_to(bdec_ref[...], out_ref.shape)

    out_ref[...] += jnp.dot(z_ref[...], w_ref[...],
                            preferred_element_type=jnp.float32)


def kernel(x, W_enc, b_enc, W_dec, b_dec):
    n_tok, d_in = x.shape
    d_sae = W_enc.shape[1]
    f32 = jnp.float32

    b_enc2 = b_enc.reshape(1, d_sae)
    b_dec2 = b_dec.reshape(1, d_in)

    tb = min(1024, n_tok)
    sb = min(1024, d_sae)
    nt, ns = n_tok // tb, d_sae // sb
    z = pl.pallas_call(
        _encode_kernel,
        grid=(ns, nt),
        in_specs=[
            pl.BlockSpec((tb, d_in), lambda j, i: (i, 0)),
            pl.BlockSpec((d_in, sb), lambda j, i: (0, j)),
            pl.BlockSpec((1, sb), lambda j, i: (0, j)),
            pl.BlockSpec((1, d_in), lambda j, i: (0, 0)),
        ],
        out_specs=pl.BlockSpec((tb, sb), lambda j, i: (i, j)),
        out_shape=jax.ShapeDtypeStruct((n_tok, d_sae), f32),
        compiler_params=pltpu.CompilerParams(
            dimension_semantics=("arbitrary", "arbitrary"),
        ),
    )(x, W_enc, b_enc2, b_dec2)

    mb = min(128, n_tok)
    z_sparse = pl.pallas_call(
        functools.partial(_mask_kernel, k=K_TOP),
        grid=(n_tok // mb,),
        in_specs=[pl.BlockSpec((mb, d_sae), lambda i: (i, 0))],
        out_specs=pl.BlockSpec((mb, d_sae), lambda i: (i, 0)),
        out_shape=jax.ShapeDtypeStruct((n_tok, d_sae), f32),
    )(z)

    tb2 = min(1024, n_tok)
    kb2 = min(1024, d_sae)
    x_rec = pl.pallas_call(
        _decode_kernel,
        grid=(n_tok // tb2, d_sae // kb2),
        in_specs=[
            pl.BlockSpec((tb2, kb2), lambda i, j: (i, j)),
            pl.BlockSpec((kb2, d_in), lambda i, j: (j, 0)),
            pl.BlockSpec((1, d_in), lambda i, j: (0, 0)),
        ],
        out_specs=pl.BlockSpec((tb2, d_in), lambda i, j: (i, 0)),
        out_shape=jax.ShapeDtypeStruct((n_tok, d_in), f32),
        compiler_params=pltpu.CompilerParams(
            dimension_semantics=("parallel", "arbitrary"),
        ),
    )(z_sparse, W_dec, b_dec2)

    return (x_rec, z_sparse)
